# Initial kernel scaffold; baseline (speedup 1.0000x reference)
import jax
import jax.numpy as jnp
from jax.experimental import pallas as pl


def kernel(x, edge_index, W1, b1, W2, b2, W3, b3, W4, b4, W5, b5, W6, b6, g1, be1, g2, be2, g3, be3, g4, be4, g5, be5):
    raise NotImplementedError("write your pallas kernel here")


# speedup vs baseline: 24.4788x; 24.4788x over previous
import functools

import jax
import jax.numpy as jnp
from jax import lax
from jax.experimental import pallas as pl
from jax.experimental.pallas import tpu as pltpu
from jax.experimental.pallas import tpu_sc as plsc

_N = 10000
_E = 320000
_EPS = 1e-5

_NC = 2
_NS = 16
_NW = _NC * _NS
_CHUNK = _E // _NW
_LANES = 16
_VECS = _CHUNK // _LANES

_BN = 1000
_NB = _N // _BN


def _sc_segsum(cols, src, dst, zeros):
  mesh = plsc.VectorSubcoreMesh(core_axis_name="c", subcore_axis_name="s")

  @functools.partial(
      pl.kernel,
      mesh=mesh,
      compiler_params=pltpu.CompilerParams(needs_layout_passes=False),
      out_type=jax.ShapeDtypeStruct((_NC, 2 * _N), jnp.float32),
      scratch_types=[
          pltpu.VMEM((2 * _N,), jnp.float32),
          pltpu.VMEM((_CHUNK,), jnp.int32),
          pltpu.VMEM((_CHUNK,), jnp.int32),
          pltpu.VMEM((2 * _CHUNK,), jnp.float32),
          pltpu.VMEM((2 * _CHUNK,), jnp.int32),
          pltpu.VMEM_SHARED((2 * _N,), jnp.float32),
      ],
  )
  def k(cols_hbm, src_hbm, dst_hbm, zeros_hbm, out_hbm,
        cols_v, src_v, dst_v, upd_v, idx_v, acc_sh):
    c = lax.axis_index("c")
    s = lax.axis_index("s")
    wid = c * _NS + s
    off = wid * _CHUNK

    @pl.when(s == 0)
    def _zero():
      pltpu.sync_copy(zeros_hbm, acc_sh)

    pltpu.sync_copy(cols_hbm, cols_v)
    pltpu.sync_copy(src_hbm.at[pl.ds(off, _CHUNK)], src_v)
    pltpu.sync_copy(dst_hbm.at[pl.ds(off, _CHUNK)], dst_v)

    lane = lax.iota(jnp.int32, _LANES)

    def body(i, carry):
      sv = src_v[pl.ds(i * _LANES, _LANES)]
      dv = dst_v[pl.ds(i * _LANES, _LANES)]
      sbase = sv + sv
      g0 = plsc.load_gather(cols_v, [sbase])
      g1 = plsc.load_gather(cols_v, [sbase + 1])
      pos = (lane + i * _LANES) * 2
      plsc.store_scatter(upd_v, [pos], g0)
      plsc.store_scatter(upd_v, [pos + 1], g1)
      dbase = dv + dv
      plsc.store_scatter(idx_v, [pos], dbase)
      plsc.store_scatter(idx_v, [pos + 1], dbase + 1)
      return carry

    lax.fori_loop(0, _VECS, body, 0)
    plsc.subcore_barrier()
    pltpu.sync_copy(upd_v, acc_sh.at[idx_v], add=True)
    plsc.subcore_barrier()

    @pl.when(s == 0)
    def _flush():
      pltpu.sync_copy(acc_sh, out_hbm.at[c])

  return k(cols, src, dst, zeros)


def _tc_layer(h, sp, cp, whT, waT, b, g, be):
  din = h.shape[1]
  dout = whT.shape[1]

  def body(h_ref, sp_ref, cp_ref, whT_ref, waT_ref, b_ref, g_ref, be_ref,
           out_ref, o_scr, s1, s2):
    p = pl.program_id(0)
    i = pl.program_id(1)

    @pl.when(p == 0)
    def _compute():
      @pl.when(i == 0)
      def _init():
        s1[...] = jnp.zeros_like(s1)
        s2[...] = jnp.zeros_like(s2)

      spb = sp_ref[...]
      cpb = cp_ref[...]
      cnt = cpb[0, :, 0:1] + cpb[1, :, 0:1]
      inv = 1.0 / jnp.maximum(cnt, 1.0)
      a = (spb[0] + spb[1]) * inv
      out = jnp.dot(h_ref[...], whT_ref[...],
                    preferred_element_type=jnp.float32)
      out = (out + a[:, 0:1] * waT_ref[0:1, :] + a[:, 1:2] * waT_ref[1:2, :]
             + b_ref[...])
      o_scr[pl.ds(i * _BN, _BN), :] = out
      s1[...] += jnp.sum(out, axis=0, keepdims=True)
      s2[...] += jnp.sum(out * out, axis=0, keepdims=True)

    @pl.when(p == 1)
    def _normalize():
      o = o_scr[pl.ds(i * _BN, _BN), :]
      mean = s1[...] * (1.0 / _N)
      var = s2[...] * (1.0 / _N) - mean * mean
      scale = g_ref[...] * lax.rsqrt(var + _EPS)
      shift = be_ref[...] - mean * scale
      out_ref[...] = jnp.maximum(o * scale + shift, 0.0)

  return pl.pallas_call(
      body,
      grid=(2, _NB),
      in_specs=[
          pl.BlockSpec((_BN, din), lambda p, i: (jnp.where(p == 0, i, 0), 0)),
          pl.BlockSpec((_NC, _BN, 2),
                       lambda p, i: (0, jnp.where(p == 0, i, 0), 0)),
          pl.BlockSpec((_NC, _BN, 2),
                       lambda p, i: (0, jnp.where(p == 0, i, 0), 0)),
          pl.BlockSpec((din, dout), lambda p, i: (0, 0)),
          pl.BlockSpec((2, dout), lambda p, i: (0, 0)),
          pl.BlockSpec((1, dout), lambda p, i: (0, 0)),
          pl.BlockSpec((1, dout), lambda p, i: (0, 0)),
          pl.BlockSpec((1, dout), lambda p, i: (0, 0)),
      ],
      out_specs=pl.BlockSpec((_BN, dout),
                             lambda p, i: (jnp.where(p == 1, i, 0), 0)),
      out_shape=jax.ShapeDtypeStruct((_N, dout), jnp.float32),
      scratch_shapes=[
          pltpu.VMEM((_N, dout), jnp.float32),
          pltpu.VMEM((1, dout), jnp.float32),
          pltpu.VMEM((1, dout), jnp.float32),
      ],
  )(h, sp, cp, whT, waT, b, g, be)


def _tc_final(h, sp, cp, whT, waT, b):
  din = h.shape[1]
  dout = whT.shape[1]

  def body(h_ref, sp_ref, cp_ref, whT_ref, waT_ref, b_ref, out_ref):
    spb = sp_ref[...]
    cpb = cp_ref[...]
    cnt = cpb[0, :, 0:1] + cpb[1, :, 0:1]
    inv = 1.0 / jnp.maximum(cnt, 1.0)
    a = (spb[0] + spb[1]) * inv
    out = jnp.dot(h_ref[...], whT_ref[...], preferred_element_type=jnp.float32)
    out_ref[...] = (out + a[:, 0:1] * waT_ref[0:1, :]
                    + a[:, 1:2] * waT_ref[1:2, :] + b_ref[...])

  return pl.pallas_call(
      body,
      grid=(_NB,),
      in_specs=[
          pl.BlockSpec((_BN, din), lambda i: (i, 0)),
          pl.BlockSpec((_NC, _BN, 2), lambda i: (0, i, 0)),
          pl.BlockSpec((_NC, _BN, 2), lambda i: (0, i, 0)),
          pl.BlockSpec((din, dout), lambda i: (0, 0)),
          pl.BlockSpec((2, dout), lambda i: (0, 0)),
          pl.BlockSpec((1, dout), lambda i: (0, 0)),
      ],
      out_specs=pl.BlockSpec((_BN, dout), lambda i: (i, 0)),
      out_shape=jax.ShapeDtypeStruct((_N, dout), jnp.float32),
  )(h, sp, cp, whT, waT, b)


def kernel(x, edge_index, W1, b1, W2, b2, W3, b3, W4, b4, W5, b5, W6, b6,
           g1, be1, g2, be2, g3, be3, g4, be4, g5, be5):
  src = edge_index[0]
  dst = edge_index[1]
  zeros = jnp.zeros((2 * _N,), jnp.float32)
  ones = jnp.ones((2 * _N,), jnp.float32)

  cp = _sc_segsum(ones, src, dst, zeros).reshape(_NC, _N, 2)

  h = x
  for W, b, g, be in ((W1, b1, g1, be1), (W2, b2, g2, be2),
                      (W3, b3, g3, be3), (W4, b4, g4, be4),
                      (W5, b5, g5, be5)):
    din = h.shape[1]
    sp = _sc_segsum(h[:, :2].reshape(-1), src, dst, zeros).reshape(_NC, _N, 2)
    h = _tc_layer(h, sp, cp, W[:, :din].T, W[:, din:].T,
                  b.reshape(1, -1), g.reshape(1, -1), be.reshape(1, -1))

  din = h.shape[1]
  sp = _sc_segsum(h[:, :2].reshape(-1), src, dst, zeros).reshape(_NC, _N, 2)
  return _tc_final(h, sp, cp, W6[:, :din].T, W6[:, din:].T, b6.reshape(1, -1))


# speedup vs baseline: 26.6372x; 1.0882x over previous
import functools

import jax
import jax.numpy as jnp
from jax import lax
from jax.experimental import pallas as pl
from jax.experimental.pallas import tpu as pltpu
from jax.experimental.pallas import tpu_sc as plsc

_N = 10000
_E = 320000
_EPS = 1e-5

_NC = 2
_NS = 16
_NW = _NC * _NS
_CHUNK = _E // _NW
_LANES = 16
_VECS = _CHUNK // _LANES

_BN = 1000
_NB = _N // _BN


def _sc_segsum(cols, edge_index, zeros):
  mesh = plsc.VectorSubcoreMesh(core_axis_name="c", subcore_axis_name="s")

  @functools.partial(
      pl.kernel,
      mesh=mesh,
      compiler_params=pltpu.CompilerParams(needs_layout_passes=False),
      out_type=jax.ShapeDtypeStruct((_NC, 2 * _N), jnp.float32),
      scratch_types=[
          pltpu.VMEM((2 * _N,), jnp.float32),
          pltpu.VMEM((_CHUNK,), jnp.int32),
          pltpu.VMEM((_CHUNK,), jnp.int32),
          pltpu.VMEM((2 * _CHUNK,), jnp.float32),
          pltpu.VMEM((2 * _CHUNK,), jnp.int32),
          pltpu.VMEM_SHARED((2 * _N,), jnp.float32),
      ],
  )
  def k(cols_hbm, ei_hbm, zeros_hbm, out_hbm,
        cols_v, src_v, dst_v, upd_v, idx_v, acc_sh):
    c = lax.axis_index("c")
    s = lax.axis_index("s")
    wid = c * _NS + s
    off = wid * _CHUNK

    @pl.when(s == 0)
    def _zero():
      pltpu.sync_copy(zeros_hbm, acc_sh)

    pltpu.sync_copy(cols_hbm, cols_v)
    pltpu.sync_copy(ei_hbm.at[pl.ds(off, _CHUNK)], src_v)
    pltpu.sync_copy(ei_hbm.at[pl.ds(_E + off, _CHUNK)], dst_v)

    lane = lax.iota(jnp.int32, _LANES)

    def body(i, carry):
      sv = src_v[pl.ds(i * _LANES, _LANES)]
      dv = dst_v[pl.ds(i * _LANES, _LANES)]
      sbase = sv + sv
      g0 = plsc.load_gather(cols_v, [sbase])
      g1 = plsc.load_gather(cols_v, [sbase + 1])
      pos = (lane + i * _LANES) * 2
      plsc.store_scatter(upd_v, [pos], g0)
      plsc.store_scatter(upd_v, [pos + 1], g1)
      dbase = dv + dv
      plsc.store_scatter(idx_v, [pos], dbase)
      plsc.store_scatter(idx_v, [pos + 1], dbase + 1)
      return carry

    lax.fori_loop(0, _VECS, body, 0)
    plsc.subcore_barrier()
    pltpu.sync_copy(upd_v, acc_sh.at[idx_v], add=True)
    plsc.subcore_barrier()

    @pl.when(s == 0)
    def _flush():
      pltpu.sync_copy(acc_sh, out_hbm.at[c])

  return k(cols, edge_index, zeros)


def _tc_layer(h, ag, cnt, whT, waT, b, g, be):
  din = h.shape[1]
  dout = whT.shape[1]

  def body(h_ref, ag_ref, cnt_ref, whT_ref, waT_ref, b_ref, g_ref, be_ref,
           out_ref, o_scr, s1, s2):
    p = pl.program_id(0)
    i = pl.program_id(1)

    @pl.when(p == 0)
    def _compute():
      @pl.when(i == 0)
      def _init():
        s1[...] = jnp.zeros_like(s1)
        s2[...] = jnp.zeros_like(s2)

      inv = 1.0 / jnp.maximum(cnt_ref[...][:, 0:1], 1.0)
      a = ag_ref[...] * inv
      out = jnp.dot(h_ref[...], whT_ref[...],
                    preferred_element_type=jnp.float32)
      out = (out + a[:, 0:1] * waT_ref[0:1, :] + a[:, 1:2] * waT_ref[1:2, :]
             + b_ref[...])
      o_scr[pl.ds(i * _BN, _BN), :] = out
      s1[...] += jnp.sum(out, axis=0, keepdims=True)
      s2[...] += jnp.sum(out * out, axis=0, keepdims=True)

    @pl.when(p == 1)
    def _normalize():
      o = o_scr[pl.ds(i * _BN, _BN), :]
      mean = s1[...] * (1.0 / _N)
      var = s2[...] * (1.0 / _N) - mean * mean
      scale = g_ref[...] * lax.rsqrt(var + _EPS)
      shift = be_ref[...] - mean * scale
      out_ref[...] = jnp.maximum(o * scale + shift, 0.0)

  return pl.pallas_call(
      body,
      grid=(2, _NB),
      in_specs=[
          pl.BlockSpec((_BN, din), lambda p, i: (jnp.where(p == 0, i, 0), 0)),
          pl.BlockSpec((_BN, 2), lambda p, i: (jnp.where(p == 0, i, 0), 0)),
          pl.BlockSpec((_BN, 2), lambda p, i: (jnp.where(p == 0, i, 0), 0)),
          pl.BlockSpec((din, dout), lambda p, i: (0, 0)),
          pl.BlockSpec((2, dout), lambda p, i: (0, 0)),
          pl.BlockSpec((1, dout), lambda p, i: (0, 0)),
          pl.BlockSpec((1, dout), lambda p, i: (0, 0)),
          pl.BlockSpec((1, dout), lambda p, i: (0, 0)),
      ],
      out_specs=pl.BlockSpec((_BN, dout),
                             lambda p, i: (jnp.where(p == 1, i, 0), 0)),
      out_shape=jax.ShapeDtypeStruct((_N, dout), jnp.float32),
      scratch_shapes=[
          pltpu.VMEM((_N, dout), jnp.float32),
          pltpu.VMEM((1, dout), jnp.float32),
          pltpu.VMEM((1, dout), jnp.float32),
      ],
  )(h, ag, cnt, whT, waT, b, g, be)


def _tc_final(h, ag, cnt, whT, waT, b):
  din = h.shape[1]
  dout = whT.shape[1]

  def body(h_ref, ag_ref, cnt_ref, whT_ref, waT_ref, b_ref, out_ref):
    inv = 1.0 / jnp.maximum(cnt_ref[...][:, 0:1], 1.0)
    a = ag_ref[...] * inv
    out = jnp.dot(h_ref[...], whT_ref[...], preferred_element_type=jnp.float32)
    out_ref[...] = (out + a[:, 0:1] * waT_ref[0:1, :]
                    + a[:, 1:2] * waT_ref[1:2, :] + b_ref[...])

  return pl.pallas_call(
      body,
      grid=(_NB,),
      in_specs=[
          pl.BlockSpec((_BN, din), lambda i: (i, 0)),
          pl.BlockSpec((_BN, 2), lambda i: (i, 0)),
          pl.BlockSpec((_BN, 2), lambda i: (i, 0)),
          pl.BlockSpec((din, dout), lambda i: (0, 0)),
          pl.BlockSpec((2, dout), lambda i: (0, 0)),
          pl.BlockSpec((1, dout), lambda i: (0, 0)),
      ],
      out_specs=pl.BlockSpec((_BN, dout), lambda i: (i, 0)),
      out_shape=jax.ShapeDtypeStruct((_N, dout), jnp.float32),
  )(h, ag, cnt, whT, waT, b)


def kernel(x, edge_index, W1, b1, W2, b2, W3, b3, W4, b4, W5, b5, W6, b6,
           g1, be1, g2, be2, g3, be3, g4, be4, g5, be5):
  zeros = jnp.zeros((2 * _N,), jnp.float32)
  ones = jnp.ones((2 * _N,), jnp.float32)
  ei = jnp.ravel(edge_index)

  cpp = _sc_segsum(ones, ei, zeros)
  cnt = (cpp[0] + cpp[1]).reshape(_N, 2)

  h = x
  cols = x[:, :2].reshape(-1)
  for W, b, g, be in ((W1, b1, g1, be1), (W2, b2, g2, be2),
                      (W3, b3, g3, be3), (W4, b4, g4, be4),
                      (W5, b5, g5, be5)):
    din = h.shape[1]
    sp = _sc_segsum(cols, ei, zeros)
    ag = (sp[0] + sp[1]).reshape(_N, 2)
    h = _tc_layer(h, ag, cnt, W[:, :din].T, W[:, din:].T,
                  b.reshape(1, -1), g.reshape(1, -1), be.reshape(1, -1))
    cols = h[:, :2].reshape(-1)

  din = h.shape[1]
  sp = _sc_segsum(cols, ei, zeros)
  ag = (sp[0] + sp[1]).reshape(_N, 2)
  return _tc_final(h, ag, cnt, W6[:, :din].T, W6[:, din:].T, b6.reshape(1, -1))


# speedup vs baseline: 26.9893x; 1.0132x over previous
import functools

import jax
import jax.numpy as jnp
from jax import lax
from jax.experimental import pallas as pl
from jax.experimental.pallas import tpu as pltpu
from jax.experimental.pallas import tpu_sc as plsc

_N = 10000
_E = 320000
_EPS = 1e-5

_NC = 2
_NS = 16
_NW = _NC * _NS
_CHUNK = _E // _NW
_LANES = 16
_VECS = _CHUNK // _LANES

_BN = 1000
_NB = _N // _BN


def _sc_segsum(cols, edge_index, zeros):
  mesh = plsc.VectorSubcoreMesh(core_axis_name="c", subcore_axis_name="s")

  @functools.partial(
      pl.kernel,
      mesh=mesh,
      compiler_params=pltpu.CompilerParams(needs_layout_passes=False),
      out_type=jax.ShapeDtypeStruct((_NC, 2 * _N), jnp.float32),
      scratch_types=[
          pltpu.VMEM((2 * _N,), jnp.float32),
          pltpu.VMEM((_CHUNK,), jnp.int32),
          pltpu.VMEM((_CHUNK,), jnp.int32),
          pltpu.VMEM((2 * _CHUNK,), jnp.float32),
          pltpu.VMEM((2 * _CHUNK,), jnp.int32),
          pltpu.VMEM_SHARED((2 * _N,), jnp.float32),
      ],
  )
  def k(cols_hbm, ei_hbm, zeros_hbm, out_hbm,
        cols_v, src_v, dst_v, upd_v, idx_v, acc_sh):
    c = lax.axis_index("c")
    s = lax.axis_index("s")
    wid = c * _NS + s
    off = wid * _CHUNK

    @pl.when(s == 0)
    def _zero():
      pltpu.sync_copy(zeros_hbm, acc_sh)

    pltpu.sync_copy(cols_hbm, cols_v)
    pltpu.sync_copy(ei_hbm.at[pl.ds(off, _CHUNK)], src_v)
    pltpu.sync_copy(ei_hbm.at[pl.ds(_E + off, _CHUNK)], dst_v)

    lane = lax.iota(jnp.int32, _LANES)

    def body(i, carry):
      sv = src_v[pl.ds(i * _LANES, _LANES)]
      dv = dst_v[pl.ds(i * _LANES, _LANES)]
      sbase = sv + sv
      g0 = plsc.load_gather(cols_v, [sbase])
      g1 = plsc.load_gather(cols_v, [sbase + 1])
      pos = (lane + i * _LANES) * 2
      plsc.store_scatter(upd_v, [pos], g0)
      plsc.store_scatter(upd_v, [pos + 1], g1)
      dbase = dv + dv
      plsc.store_scatter(idx_v, [pos], dbase)
      plsc.store_scatter(idx_v, [pos + 1], dbase + 1)
      return carry

    lax.fori_loop(0, _VECS, body, 0)
    plsc.subcore_barrier()
    pltpu.sync_copy(upd_v, acc_sh.at[idx_v], add=True)
    plsc.subcore_barrier()

    @pl.when(s == 0)
    def _flush():
      pltpu.sync_copy(acc_sh, out_hbm.at[c])

  return k(cols, edge_index, zeros)


def _tc_layer(h, ag, cnt, whT, waT, b, g, be):
  din = h.shape[1]
  dout = whT.shape[1]

  def body(h_ref, ag_ref, cnt_ref, whT_ref, waT_ref, b_ref, g_ref, be_ref,
           out_ref, o_scr, s1, s2):
    p = pl.program_id(0)
    i = pl.program_id(1)

    @pl.when(p == 0)
    def _compute():
      @pl.when(i == 0)
      def _init():
        s1[...] = jnp.zeros_like(s1)
        s2[...] = jnp.zeros_like(s2)

      inv = 1.0 / jnp.maximum(cnt_ref[...][:, 0:1], 1.0)
      a = ag_ref[...] * inv
      out = jnp.dot(h_ref[...], whT_ref[...],
                    preferred_element_type=jnp.float32)
      out = (out + a[:, 0:1] * waT_ref[0:1, :] + a[:, 1:2] * waT_ref[1:2, :]
             + b_ref[...])
      o_scr[pl.ds(i * _BN, _BN), :] = out
      s1[...] += jnp.sum(out, axis=0, keepdims=True)
      s2[...] += jnp.sum(out * out, axis=0, keepdims=True)

    @pl.when(p == 1)
    def _normalize():
      o = o_scr[pl.ds(i * _BN, _BN), :]
      mean = s1[...] * (1.0 / _N)
      var = s2[...] * (1.0 / _N) - mean * mean
      scale = g_ref[...] * lax.rsqrt(var + _EPS)
      shift = be_ref[...] - mean * scale
      out_ref[...] = jnp.maximum(o * scale + shift, 0.0)

  return pl.pallas_call(
      body,
      grid=(2, _NB),
      in_specs=[
          pl.BlockSpec((_BN, din), lambda p, i: (jnp.where(p == 0, i, 0), 0)),
          pl.BlockSpec((_BN, 2), lambda p, i: (jnp.where(p == 0, i, 0), 0)),
          pl.BlockSpec((_BN, 2), lambda p, i: (jnp.where(p == 0, i, 0), 0)),
          pl.BlockSpec((din, dout), lambda p, i: (0, 0)),
          pl.BlockSpec((2, dout), lambda p, i: (0, 0)),
          pl.BlockSpec((1, dout), lambda p, i: (0, 0)),
          pl.BlockSpec((1, dout), lambda p, i: (0, 0)),
          pl.BlockSpec((1, dout), lambda p, i: (0, 0)),
      ],
      out_specs=pl.BlockSpec((_BN, dout),
                             lambda p, i: (jnp.where(p == 1, i, 0), 0)),
      out_shape=jax.ShapeDtypeStruct((_N, dout), jnp.float32),
      scratch_shapes=[
          pltpu.VMEM((_N, dout), jnp.float32),
          pltpu.VMEM((1, dout), jnp.float32),
          pltpu.VMEM((1, dout), jnp.float32),
      ],
  )(h, ag, cnt, whT, waT, b, g, be)


def _tc_cols2(h, ag, cnt, whT2, waT2, b2, g2, be2):
  din = h.shape[1]

  def body(h_ref, ag_ref, cnt_ref, whT2_ref, waT2_ref, b2_ref, g2_ref,
           be2_ref, cols_ref, o_scr, s1, s2):
    p = pl.program_id(0)
    i = pl.program_id(1)

    @pl.when(p == 0)
    def _compute():
      @pl.when(i == 0)
      def _init():
        s1[...] = jnp.zeros_like(s1)
        s2[...] = jnp.zeros_like(s2)

      inv = 1.0 / jnp.maximum(cnt_ref[...][:, 0:1], 1.0)
      a = ag_ref[...] * inv
      o2 = jnp.dot(h_ref[...], whT2_ref[...],
                   preferred_element_type=jnp.float32)
      o2 = (o2 + a[:, 0:1] * waT2_ref[0:1, :] + a[:, 1:2] * waT2_ref[1:2, :]
            + b2_ref[...])
      o_scr[pl.ds(i * _BN, _BN), :] = o2
      s1[...] += jnp.sum(o2, axis=0, keepdims=True)
      s2[...] += jnp.sum(o2 * o2, axis=0, keepdims=True)

    @pl.when(p == 1)
    def _normalize():
      o = o_scr[pl.ds(i * _BN, _BN), :]
      mean = s1[...] * (1.0 / _N)
      var = s2[...] * (1.0 / _N) - mean * mean
      scale = g2_ref[...] * lax.rsqrt(var + _EPS)
      shift = be2_ref[...] - mean * scale
      cols_ref[...] = jnp.maximum(o * scale + shift, 0.0)

  return pl.pallas_call(
      body,
      grid=(2, _NB),
      in_specs=[
          pl.BlockSpec((_BN, din), lambda p, i: (jnp.where(p == 0, i, 0), 0)),
          pl.BlockSpec((_BN, 2), lambda p, i: (jnp.where(p == 0, i, 0), 0)),
          pl.BlockSpec((_BN, 2), lambda p, i: (jnp.where(p == 0, i, 0), 0)),
          pl.BlockSpec((din, 2), lambda p, i: (0, 0)),
          pl.BlockSpec((2, 2), lambda p, i: (0, 0)),
          pl.BlockSpec((1, 2), lambda p, i: (0, 0)),
          pl.BlockSpec((1, 2), lambda p, i: (0, 0)),
          pl.BlockSpec((1, 2), lambda p, i: (0, 0)),
      ],
      out_specs=pl.BlockSpec((_BN, 2),
                             lambda p, i: (jnp.where(p == 1, i, 0), 0)),
      out_shape=jax.ShapeDtypeStruct((_N, 2), jnp.float32),
      scratch_shapes=[
          pltpu.VMEM((_N, 2), jnp.float32),
          pltpu.VMEM((1, 2), jnp.float32),
          pltpu.VMEM((1, 2), jnp.float32),
      ],
  )(h, ag, cnt, whT2, waT2, b2, g2, be2)


def _tc_final(h, ag, cnt, whT, waT, b):
  din = h.shape[1]
  dout = whT.shape[1]

  def body(h_ref, ag_ref, cnt_ref, whT_ref, waT_ref, b_ref, out_ref):
    inv = 1.0 / jnp.maximum(cnt_ref[...][:, 0:1], 1.0)
    a = ag_ref[...] * inv
    out = jnp.dot(h_ref[...], whT_ref[...], preferred_element_type=jnp.float32)
    out_ref[...] = (out + a[:, 0:1] * waT_ref[0:1, :]
                    + a[:, 1:2] * waT_ref[1:2, :] + b_ref[...])

  return pl.pallas_call(
      body,
      grid=(_NB,),
      in_specs=[
          pl.BlockSpec((_BN, din), lambda i: (i, 0)),
          pl.BlockSpec((_BN, 2), lambda i: (i, 0)),
          pl.BlockSpec((_BN, 2), lambda i: (i, 0)),
          pl.BlockSpec((din, dout), lambda i: (0, 0)),
          pl.BlockSpec((2, dout), lambda i: (0, 0)),
          pl.BlockSpec((1, dout), lambda i: (0, 0)),
      ],
      out_specs=pl.BlockSpec((_BN, dout), lambda i: (i, 0)),
      out_shape=jax.ShapeDtypeStruct((_N, dout), jnp.float32),
  )(h, ag, cnt, whT, waT, b)


def kernel(x, edge_index, W1, b1, W2, b2, W3, b3, W4, b4, W5, b5, W6, b6,
           g1, be1, g2, be2, g3, be3, g4, be4, g5, be5):
  zeros = jnp.zeros((2 * _N,), jnp.float32)
  ones = jnp.ones((2 * _N,), jnp.float32)
  ei = jnp.ravel(edge_index)

  cpp = _sc_segsum(ones, ei, zeros)
  cnt = (cpp[0] + cpp[1]).reshape(_N, 2)

  h = x
  cols = x[:, :2].reshape(-1)
  for W, b, g, be in ((W1, b1, g1, be1), (W2, b2, g2, be2),
                      (W3, b3, g3, be3), (W4, b4, g4, be4),
                      (W5, b5, g5, be5)):
    din = h.shape[1]
    sp = _sc_segsum(cols, ei, zeros)
    ag = (sp[0] + sp[1]).reshape(_N, 2)
    cols = _tc_cols2(h, ag, cnt, W[0:2, :din].T, W[0:2, din:].T,
                     b[0:2].reshape(1, -1), g[0:2].reshape(1, -1),
                     be[0:2].reshape(1, -1)).reshape(-1)
    h = _tc_layer(h, ag, cnt, W[:, :din].T, W[:, din:].T,
                  b.reshape(1, -1), g.reshape(1, -1), be.reshape(1, -1))

  din = h.shape[1]
  sp = _sc_segsum(cols, ei, zeros)
  ag = (sp[0] + sp[1]).reshape(_N, 2)
  return _tc_final(h, ag, cnt, W6[:, :din].T, W6[:, din:].T, b6.reshape(1, -1))
